# baseline (speedup 1.0000x reference)
import functools

import jax
import jax.numpy as jnp
from jax import lax
from jax.experimental import pallas as pl
from jax.experimental.pallas import tpu as pltpu
from jax.experimental.pallas import tpu_sc as plsc

B, C, N, K = 1, 128, 10000, 32
COUT = 128
NK = N * K

NC, NS = 2, 16
NW = NC * NS
NP = 320
NPAD = NW * NP
NT = NPAD


def _mm_body(xst_ref, wt_ref, tab_ref):
    xsb = xst_ref[...]
    wt = wt_ref[...]
    at = wt[:C, :] - wt[C:, :]
    tab_ref[:NT, :] = jnp.dot(xsb, at, preferred_element_type=jnp.float32)
    tab_ref[NT:, :] = jnp.dot(xsb, wt[C:, :], preferred_element_type=jnp.float32)


def _sc_body(tab_hbm, idx_hbm,
             mx_hbm, sums_hbm, sumsqs_hbm,
             idx_v, buf_v, outb_v, s_v, ss_v,
             gsem0, gsem1, osem0, osem1):
    wid = lax.axis_index("s") * NC + lax.axis_index("c")
    base = wid * NP
    gsem = (gsem0, gsem1)
    osem = (osem0, osem1)

    pltpu.sync_copy(idx_hbm.at[wid], idx_v)

    for b in range(2):
        pltpu.async_copy(tab_hbm.at[idx_v.at[b]], buf_v.at[b], gsem[b])

    zero = jnp.zeros((16,), jnp.float32)
    init = tuple(zero for _ in range(2 * (C // 16)))

    def big_body(G, carry):
        acc0 = carry
        for h in range(2):
            @pl.when(G >= 1)
            def _():
                pltpu.make_async_copy(
                    outb_v.at[h],
                    mx_hbm.at[pl.ds(base + (G - 1) * 32 + h * 16, 16)],
                    osem[h]).wait()

            def pair_body(j, carry2):
                acc = list(carry2)
                for b in range(2):
                    t = G * 32 + h * 16 + j * 2 + b
                    pltpu.make_async_copy(
                        tab_hbm.at[idx_v.at[t]], buf_v.at[b], gsem[b]).wait()
                    for c in range(C // 16):
                        sl = pl.ds(c * 16, 16)
                        mx = jnp.full((16,), -jnp.inf, jnp.float32)
                        mn = jnp.full((16,), jnp.inf, jnp.float32)
                        s = acc[2 * c]
                        ss = acc[2 * c + 1]
                        for k in range(K):
                            y = buf_v[b, k, sl] + buf_v[b, K + k, sl]
                            mx = jnp.maximum(mx, y)
                            mn = jnp.minimum(mn, y)
                            s = s + y
                            ss = ss + y * y
                        acc[2 * c] = s
                        acc[2 * c + 1] = ss
                        outb_v[h, j * 2 + b, 0, sl] = mx
                        outb_v[h, j * 2 + b, 1, sl] = mn

                    @pl.when(t + 2 < NP)
                    def _():
                        pltpu.async_copy(
                            tab_hbm.at[idx_v.at[t + 2]], buf_v.at[b], gsem[b])
                return tuple(acc)

            acc0 = lax.fori_loop(0, 8, pair_body, acc0)
            pltpu.async_copy(
                outb_v.at[h],
                mx_hbm.at[pl.ds(base + G * 32 + h * 16, 16)],
                osem[h])
        return acc0

    acc = lax.fori_loop(0, NP // 32, big_body, init)

    for h in range(2):
        pltpu.make_async_copy(
            outb_v.at[h],
            mx_hbm.at[pl.ds(base + NP - 32 + h * 16, 16)],
            osem[h]).wait()

    for c in range(C // 16):
        sl = pl.ds(c * 16, 16)
        s_v[sl] = acc[2 * c]
        ss_v[sl] = acc[2 * c + 1]
    pltpu.sync_copy(s_v, sums_hbm.at[wid])
    pltpu.sync_copy(ss_v, sumsqs_hbm.at[wid])


def _fin_body(mx_ref, sums_ref, sumsqs_ref, g_ref, b_ref, out_ref):
    s = jnp.sum(sums_ref[...], axis=0, keepdims=True)
    ss = jnp.sum(sumsqs_ref[...], axis=0, keepdims=True)
    mean = s / NK
    var = ss / NK - mean * mean
    a = g_ref[...] * lax.rsqrt(var + 1e-5)
    b = b_ref[...] - a * mean
    sel = jnp.where(a >= 0, mx_ref[:, 0, :], mx_ref[:, 1, :])
    res = jnp.maximum(sel * a + b, 0.0)
    out_ref[...] = res.T


def kernel(x, edge_index, W, gamma, beta):
    f32 = jnp.float32
    xst = jnp.pad(x.reshape(C, N).T, ((0, NT - N), (0, 0)))
    wt = W.T

    tab = pl.pallas_call(
        _mm_body,
        out_shape=jax.ShapeDtypeStruct((2 * NT, COUT), f32),
    )(xst, wt)

    ii = edge_index[1].reshape(N, K).astype(jnp.int32)
    jj = edge_index[0].reshape(N, K).astype(jnp.int32)
    idx = jnp.concatenate([ii, jj + NT], axis=1)
    idx = jnp.pad(idx, ((0, NPAD - N), (0, 0)), constant_values=N)
    idx = idx.reshape(NW, NP, 2 * K)

    mesh = plsc.VectorSubcoreMesh(core_axis_name="c", subcore_axis_name="s")
    sc_fn = functools.partial(
        pl.kernel,
        mesh=mesh,
        out_type=[
            jax.ShapeDtypeStruct((NPAD, 2, COUT), f32),
            jax.ShapeDtypeStruct((NW, COUT), f32),
            jax.ShapeDtypeStruct((NW, COUT), f32),
        ],
        scratch_types=[
            pltpu.VMEM((NP, 2 * K), jnp.int32),
            pltpu.VMEM((2, 2 * K, COUT), f32),
            pltpu.VMEM((2, 16, 2, COUT), f32),
            pltpu.VMEM((COUT,), f32),
            pltpu.VMEM((COUT,), f32),
        ] + [pltpu.SemaphoreType.DMA] * 4,
    )(_sc_body)
    mx, sums, sumsqs = sc_fn(tab, idx)

    out = pl.pallas_call(
        _fin_body,
        out_shape=jax.ShapeDtypeStruct((COUT, NPAD), f32),
    )(mx, sums, sumsqs, gamma.reshape(1, COUT), beta.reshape(1, COUT))

    return out[:, :N].reshape(B, COUT, N, 1)


# speedup vs baseline: 2.4569x; 2.4569x over previous
import functools

import jax
import jax.numpy as jnp
from jax import lax
from jax.experimental import pallas as pl
from jax.experimental.pallas import tpu as pltpu
from jax.experimental.pallas import tpu_sc as plsc

B, C, N, K = 1, 128, 10000, 32
COUT = 128
NK = N * K

NC, NS = 2, 16
NW = NC * NS
NP = 314
NPAD = NW * NP
NT = NPAD


def _mm_body(xst_ref, wt_ref, tab_ref):
    xsb = xst_ref[...]
    wt = wt_ref[...]
    at = wt[:C, :] - wt[C:, :]
    tab_ref[:NT, :] = jnp.dot(xsb, at, preferred_element_type=jnp.float32)
    tab_ref[NT:, :] = jnp.dot(xsb, wt[C:, :], preferred_element_type=jnp.float32)


def _sc_body(tab_hbm, idx_hbm,
             mx_hbm, sums_hbm, sumsqs_hbm,
             idx_v, buf_v, outb_v, s_v, ss_v,
             gsem0, gsem1, osem0, osem1):
    wid = lax.axis_index("s") * NC + lax.axis_index("c")
    base = wid * NP
    gsem = (gsem0, gsem1)
    osem = (osem0, osem1)

    pltpu.sync_copy(idx_hbm.at[wid], idx_v)

    for b in range(2):
        pltpu.async_copy(tab_hbm.at[idx_v.at[b]], buf_v.at[b], gsem[b])

    zero = jnp.zeros((16,), jnp.float32)
    init = tuple(zero for _ in range(2 * (C // 16)))

    def ring_body(g, carry):
        acc = list(carry)
        for b in range(2):
            t = g * 2 + b
            n = base + t
            pltpu.make_async_copy(
                tab_hbm.at[idx_v.at[t]], buf_v.at[b], gsem[b]).wait()

            @pl.when(t >= 2)
            def _():
                pltpu.make_async_copy(
                    outb_v.at[b], mx_hbm.at[n - 2], osem[b]).wait()

            for c in range(C // 16):
                sl = pl.ds(c * 16, 16)
                mx = jnp.full((16,), -jnp.inf, jnp.float32)
                mn = jnp.full((16,), jnp.inf, jnp.float32)
                s = acc[2 * c]
                ss = acc[2 * c + 1]
                for k in range(K):
                    y = buf_v[b, k, sl] + buf_v[b, K + k, sl]
                    mx = jnp.maximum(mx, y)
                    mn = jnp.minimum(mn, y)
                    s = s + y
                    ss = ss + y * y
                acc[2 * c] = s
                acc[2 * c + 1] = ss
                outb_v[b, 0, sl] = mx
                outb_v[b, 1, sl] = mn
            pltpu.async_copy(outb_v.at[b], mx_hbm.at[n], osem[b])

            @pl.when(t + 2 < NP)
            def _():
                pltpu.async_copy(
                    tab_hbm.at[idx_v.at[t + 2]], buf_v.at[b], gsem[b])
        return tuple(acc)

    acc = lax.fori_loop(0, NP // 2, ring_body, init)

    for b in range(2):
        pltpu.make_async_copy(
            outb_v.at[b], mx_hbm.at[base + NP - 2 + b], osem[b]).wait()

    for c in range(C // 16):
        sl = pl.ds(c * 16, 16)
        s_v[sl] = acc[2 * c]
        ss_v[sl] = acc[2 * c + 1]
    pltpu.sync_copy(s_v, sums_hbm.at[wid])
    pltpu.sync_copy(ss_v, sumsqs_hbm.at[wid])


def _fin_body(mx_ref, sums_ref, sumsqs_ref, g_ref, b_ref, out_ref):
    s = jnp.sum(sums_ref[...], axis=0, keepdims=True)
    ss = jnp.sum(sumsqs_ref[...], axis=0, keepdims=True)
    mean = s / NK
    var = ss / NK - mean * mean
    a = g_ref[...] * lax.rsqrt(var + 1e-5)
    b = b_ref[...] - a * mean
    sel = jnp.where(a >= 0, mx_ref[:, 0, :], mx_ref[:, 1, :])
    res = jnp.maximum(sel * a + b, 0.0)
    out_ref[...] = res.T


def kernel(x, edge_index, W, gamma, beta):
    f32 = jnp.float32
    xst = jnp.pad(x.reshape(C, N).T, ((0, NT - N), (0, 0)))
    wt = W.T

    tab = pl.pallas_call(
        _mm_body,
        out_shape=jax.ShapeDtypeStruct((2 * NT, COUT), f32),
    )(xst, wt)

    ii = edge_index[1].reshape(N, K).astype(jnp.int32)
    jj = edge_index[0].reshape(N, K).astype(jnp.int32)
    idx = jnp.concatenate([ii, jj + NT], axis=1)
    idx = jnp.pad(idx, ((0, NPAD - N), (0, 0)), constant_values=N)
    idx = idx.reshape(NW, NP, 2 * K)

    mesh = plsc.VectorSubcoreMesh(core_axis_name="c", subcore_axis_name="s")
    sc_fn = functools.partial(
        pl.kernel,
        mesh=mesh,
        out_type=[
            jax.ShapeDtypeStruct((NPAD, 2, COUT), f32),
            jax.ShapeDtypeStruct((NW, COUT), f32),
            jax.ShapeDtypeStruct((NW, COUT), f32),
        ],
        scratch_types=[
            pltpu.VMEM((NP, 2 * K), jnp.int32),
            pltpu.VMEM((2, 2 * K, COUT), f32),
            pltpu.VMEM((2, 2, COUT), f32),
            pltpu.VMEM((COUT,), f32),
            pltpu.VMEM((COUT,), f32),
        ] + [pltpu.SemaphoreType.DMA] * 4,
    )(_sc_body)
    mx, sums, sumsqs = sc_fn(tab, idx)

    out = pl.pallas_call(
        _fin_body,
        out_shape=jax.ShapeDtypeStruct((COUT, NPAD), f32),
    )(mx, sums, sumsqs, gamma.reshape(1, COUT), beta.reshape(1, COUT))

    return out[:, :N].reshape(B, COUT, N, 1)
